# Initial kernel scaffold; baseline (speedup 1.0000x reference)
import jax
import jax.numpy as jnp
from jax.experimental import pallas as pl


def kernel(x, edge_index, edge_attr, W_l1, b_l1, W_r1, b_r1, W_e1, att1, bias1, W_l2, b_l2, W_r2, b_r2, W_e2, att2, bias2):
    raise NotImplementedError("write your pallas kernel here")


# speedup vs baseline: 7.2430x; 7.2430x over previous
import functools

import jax
import jax.numpy as jnp
from jax.experimental import pallas as pl
from jax.experimental.pallas import tpu as pltpu

_N = 10000
_E = 320000
_NT = 10016
_EP = 32 * 128 * 79

_f32 = jnp.float32


def _edge_mm_body(ea_ref, we1_ref, we2_ref, e1_ref, e2_ref):
    ea = ea_ref[...]
    e1_ref[...] = jnp.dot(ea, we1_ref[...], preferred_element_type=_f32)
    e2_ref[...] = jnp.dot(ea, we2_ref[...], preferred_element_type=_f32)


def _edge_mm(eap, We1, We2):
    B = 2048
    return pl.pallas_call(
        _edge_mm_body,
        grid=(_EP // B,),
        in_specs=[
            pl.BlockSpec((B, 16), lambda i: (i, 0)),
            pl.BlockSpec((16, 64), lambda i: (0, 0)),
            pl.BlockSpec((16, 64), lambda i: (0, 0)),
        ],
        out_specs=[
            pl.BlockSpec((B, 64), lambda i: (i, 0)),
            pl.BlockSpec((B, 64), lambda i: (i, 0)),
        ],
        out_shape=[jax.ShapeDtypeStruct((_EP, 64), _f32)] * 2,
    )(eap, We1, We2)


def _prep1_body(x_ref, a0_ref, a1_ref, wl_ref, bl_ref, wr_ref, br_ref,
                we_ref, a1m_ref, xl_ref, g_ref, la_ref):
    x = x_ref[...]
    a = a0_ref[...] + a1_ref[...]
    la = a[:, :16] / jnp.maximum(a[:, 16:17], 1.0)
    la_ref[...] = la
    xl = jnp.dot(x, wl_ref[...], preferred_element_type=_f32) + bl_ref[...]
    xr = jnp.dot(x, wr_ref[...], preferred_element_type=_f32) + br_ref[...]
    el = jnp.dot(la, we_ref[...], preferred_element_type=_f32)
    m = xl + xr + el
    m = jnp.where(m >= 0, m, 0.2 * m)
    s = jnp.dot(m, a1m_ref[...], preferred_element_type=_f32)
    xl_ref[...] = xl
    g_ref[...] = jnp.concatenate(
        [xr, s, jnp.zeros((x.shape[0], 8), _f32)], axis=1)


def _prep1(x, acc0, acc1, Wl, bl, Wr, br, We, A1):
    B = 1000
    return pl.pallas_call(
        _prep1_body,
        grid=(_N // B,),
        in_specs=[
            pl.BlockSpec((B, 128), lambda i: (i, 0)),
            pl.BlockSpec((B, 32), lambda i: (i, 0)),
            pl.BlockSpec((B, 32), lambda i: (i, 0)),
            pl.BlockSpec((128, 64), lambda i: (0, 0)),
            pl.BlockSpec((1, 64), lambda i: (0, 0)),
            pl.BlockSpec((128, 64), lambda i: (0, 0)),
            pl.BlockSpec((1, 64), lambda i: (0, 0)),
            pl.BlockSpec((16, 64), lambda i: (0, 0)),
            pl.BlockSpec((64, 8), lambda i: (0, 0)),
        ],
        out_specs=[
            pl.BlockSpec((B, 64), lambda i: (i, 0)),
            pl.BlockSpec((B, 80), lambda i: (i, 0)),
            pl.BlockSpec((B, 16), lambda i: (i, 0)),
        ],
        out_shape=[
            jax.ShapeDtypeStruct((_N, 64), _f32),
            jax.ShapeDtypeStruct((_N, 80), _f32),
            jax.ShapeDtypeStruct((_N, 16), _f32),
        ],
    )(x, acc0, acc1, Wl, bl, Wr, br, We, A1)


def _msg1_body(gxl_ref, gxr_ref, e_ref, am_ref, rm_ref, out_ref):
    gxl = gxl_ref[...]
    gxr = gxr_ref[...]
    m = gxl + gxr[:, :64] + e_ref[...]
    m = jnp.where(m >= 0, m, 0.2 * m)
    alpha = jnp.dot(m, am_ref[...], preferred_element_type=_f32)
    ex = jnp.exp(jnp.clip(alpha - gxr[:, 64:72], -80.0, 75.0))
    exrep = jnp.dot(ex, rm_ref[...], preferred_element_type=_f32)
    out_ref[...] = jnp.concatenate(
        [gxl * exrep, ex, jnp.zeros((gxl.shape[0], 8), _f32)], axis=1)


def _msg1(gxl, gxr, e, A1, R8):
    B = 2048
    return pl.pallas_call(
        _msg1_body,
        grid=(_EP // B,),
        in_specs=[
            pl.BlockSpec((B, 64), lambda i: (i, 0)),
            pl.BlockSpec((B, 80), lambda i: (i, 0)),
            pl.BlockSpec((B, 64), lambda i: (i, 0)),
            pl.BlockSpec((64, 8), lambda i: (0, 0)),
            pl.BlockSpec((8, 64), lambda i: (0, 0)),
        ],
        out_specs=pl.BlockSpec((B, 80), lambda i: (i, 0)),
        out_shape=jax.ShapeDtypeStruct((_EP, 80), _f32),
    )(gxl, gxr, e, A1, R8)


def _fin1prep2_body(a0_ref, a1_ref, xl1_ref, b1_ref, rm_ref, la_ref,
                    wl_ref, bl_ref, wr_ref, br_ref, we_ref, att_ref,
                    xl2_ref, g2_ref):
    a0 = a0_ref[...]
    a1 = a1_ref[...]
    num = a0[:, :64] + a1[:, :64] + xl1_ref[...]
    den = a0[:, 64:72] + a1[:, 64:72] + 1.0
    denrep = jnp.dot(den, rm_ref[...], preferred_element_type=_f32)
    o = num / (denrep + 1e-16) + b1_ref[...]
    h = jnp.where(o > 0, o, jnp.exp(jnp.minimum(o, 0.0)) - 1.0)
    xl2 = jnp.dot(h, wl_ref[...], preferred_element_type=_f32) + bl_ref[...]
    xr2 = jnp.dot(h, wr_ref[...], preferred_element_type=_f32) + br_ref[...]
    el2 = jnp.dot(la_ref[...], we_ref[...], preferred_element_type=_f32)
    m = xl2 + xr2 + el2
    m = jnp.where(m >= 0, m, 0.2 * m)
    s2 = jnp.sum(m * att_ref[...], axis=1, keepdims=True)
    xl2_ref[...] = xl2
    g2_ref[...] = jnp.concatenate(
        [xr2, s2, jnp.zeros((xl2.shape[0], 15), _f32)], axis=1)


def _fin1prep2(acc0, acc1, XL1, bias1, R8, la, Wl2, bl2, Wr2, br2, We2, att2):
    B = 1000
    return pl.pallas_call(
        _fin1prep2_body,
        grid=(_N // B,),
        in_specs=[
            pl.BlockSpec((B, 80), lambda i: (i, 0)),
            pl.BlockSpec((B, 80), lambda i: (i, 0)),
            pl.BlockSpec((B, 64), lambda i: (i, 0)),
            pl.BlockSpec((1, 64), lambda i: (0, 0)),
            pl.BlockSpec((8, 64), lambda i: (0, 0)),
            pl.BlockSpec((B, 16), lambda i: (i, 0)),
            pl.BlockSpec((64, 64), lambda i: (0, 0)),
            pl.BlockSpec((1, 64), lambda i: (0, 0)),
            pl.BlockSpec((64, 64), lambda i: (0, 0)),
            pl.BlockSpec((1, 64), lambda i: (0, 0)),
            pl.BlockSpec((16, 64), lambda i: (0, 0)),
            pl.BlockSpec((1, 64), lambda i: (0, 0)),
        ],
        out_specs=[
            pl.BlockSpec((B, 64), lambda i: (i, 0)),
            pl.BlockSpec((B, 80), lambda i: (i, 0)),
        ],
        out_shape=[
            jax.ShapeDtypeStruct((_N, 64), _f32),
            jax.ShapeDtypeStruct((_N, 80), _f32),
        ],
    )(acc0, acc1, XL1, bias1, R8, la, Wl2, bl2, Wr2, br2, We2, att2)


def _msg2_body(gxl_ref, gxr_ref, e_ref, att_ref, out_ref):
    gxl = gxl_ref[...]
    gxr = gxr_ref[...]
    m = gxl + gxr[:, :64] + e_ref[...]
    m = jnp.where(m >= 0, m, 0.2 * m)
    alpha = jnp.sum(m * att_ref[...], axis=1, keepdims=True)
    ex = jnp.exp(jnp.clip(alpha - gxr[:, 64:65], -80.0, 75.0))
    out_ref[...] = jnp.concatenate(
        [gxl * ex, ex, jnp.zeros((gxl.shape[0], 15), _f32)], axis=1)


def _msg2(gxl, gxr, e, att2):
    B = 2048
    return pl.pallas_call(
        _msg2_body,
        grid=(_EP // B,),
        in_specs=[
            pl.BlockSpec((B, 64), lambda i: (i, 0)),
            pl.BlockSpec((B, 80), lambda i: (i, 0)),
            pl.BlockSpec((B, 64), lambda i: (i, 0)),
            pl.BlockSpec((1, 64), lambda i: (0, 0)),
        ],
        out_specs=pl.BlockSpec((B, 80), lambda i: (i, 0)),
        out_shape=jax.ShapeDtypeStruct((_EP, 80), _f32),
    )(gxl, gxr, e, att2)


def _fin2_body(a0_ref, a1_ref, xl2_ref, b2_ref, out_ref):
    a0 = a0_ref[...]
    a1 = a1_ref[...]
    num = a0[:, :64] + a1[:, :64] + xl2_ref[...]
    den = a0[:, 64:65] + a1[:, 64:65] + 1.0
    out_ref[...] = num / (den + 1e-16) + b2_ref[...]


def _fin2(acc0, acc1, XL2, bias2):
    B = 1000
    return pl.pallas_call(
        _fin2_body,
        grid=(_N // B,),
        in_specs=[
            pl.BlockSpec((B, 80), lambda i: (i, 0)),
            pl.BlockSpec((B, 80), lambda i: (i, 0)),
            pl.BlockSpec((B, 64), lambda i: (i, 0)),
            pl.BlockSpec((1, 64), lambda i: (0, 0)),
        ],
        out_specs=pl.BlockSpec((B, 64), lambda i: (i, 0)),
        out_shape=jax.ShapeDtypeStruct((_N, 64), _f32),
    )(acc0, acc1, XL2, bias2)


def _seg_add(rows, dst):
    acc = jax.ops.segment_sum(rows, dst, num_segments=_NT)
    return acc, jnp.zeros_like(acc)


def kernel(x, edge_index, edge_attr, W_l1, b_l1, W_r1, b_r1, W_e1, att1,
           bias1, W_l2, b_l2, W_r2, b_r2, W_e2, att2, bias2):
    pad = _EP - _E
    srcp = jnp.concatenate([edge_index[0], jnp.zeros((pad,), jnp.int32)])
    dstp = jnp.concatenate([edge_index[1],
                            jnp.full((pad,), _N, jnp.int32)])
    eap = jnp.concatenate([edge_attr, jnp.zeros((pad, 16), _f32)], axis=0)

    A1 = (att1[:, :, None] * jnp.eye(8, dtype=_f32)[:, None, :]).reshape(64, 8)
    R8 = jnp.repeat(jnp.eye(8, dtype=_f32), 8, axis=1)
    att2r = att2.reshape(1, 64)

    b_l1r = b_l1.reshape(1, 64)
    b_r1r = b_r1.reshape(1, 64)
    bias1r = bias1.reshape(1, 64)
    b_l2r = b_l2.reshape(1, 64)
    b_r2r = b_r2.reshape(1, 64)
    bias2r = bias2.reshape(1, 64)

    msg0 = jnp.concatenate(
        [eap, jnp.ones((_EP, 1), _f32), jnp.zeros((_EP, 15), _f32)], axis=1)
    acc00, acc01 = _seg_add(msg0, dstp)

    e1, e2 = _edge_mm(eap, W_e1, W_e2)

    XL1, G1, la = _prep1(x, acc00[:_N], acc01[:_N], W_l1, b_l1r, W_r1, b_r1r,
                         W_e1, A1)
    gxl1 = XL1[srcp]
    gxr1 = G1[dstp]
    m1 = _msg1(gxl1, gxr1, e1, A1, R8)
    acc10, acc11 = _seg_add(m1, dstp)

    XL2, G2 = _fin1prep2(acc10[:_N], acc11[:_N], XL1, bias1r, R8, la,
                         W_l2, b_l2r, W_r2, b_r2r, W_e2, att2r)
    gxl2 = XL2[srcp]
    gxr2 = G2[dstp]
    m2 = _msg2(gxl2, gxr2, e2, att2r)
    acc20, acc21 = _seg_add(m2, dstp)

    return _fin2(acc20[:_N], acc21[:_N], XL2, bias2r)
